# Initial kernel scaffold; baseline (speedup 1.0000x reference)
import functools

import jax
import jax.numpy as jnp
from jax import lax
from jax.experimental import pallas as pl
from jax.experimental.pallas import tpu as pltpu
from jax.experimental.pallas import tpu_sc as plsc

D_MODEL = 64
MAX_LEN = 16
BATCH = 16384
SEQ = 200

B_TOTAL = BATCH * SEQ
NC, NS = 2, 16
NW = NC * NS
B_PER_W = B_TOTAL // NW
CHUNK = 512
N_CHUNKS = B_PER_W // CHUNK


def _make_kernel():
    mesh = plsc.VectorSubcoreMesh(core_axis_name="c", subcore_axis_name="s")

    @functools.partial(
        pl.kernel,
        mesh=mesh,
        out_type=jax.ShapeDtypeStruct((B_TOTAL, D_MODEL), jnp.float32),
        scratch_types=[
            pltpu.VMEM((CHUNK,), jnp.int32),
            pltpu.VMEM((CHUNK, D_MODEL), jnp.float32),
            pltpu.SemaphoreType.DMA,
        ],
    )
    def emb(x_hbm, lut_hbm, out_hbm, idx_v, rows_v, sem):
        wid = lax.axis_index("s") * NC + lax.axis_index("c")
        base0 = wid * B_PER_W

        def step(i, _):
            base = base0 + i * CHUNK
            pltpu.sync_copy(x_hbm.at[pl.ds(base, CHUNK)], idx_v)
            pltpu.async_copy(lut_hbm.at[idx_v], rows_v, sem).wait()
            pltpu.sync_copy(rows_v, out_hbm.at[pl.ds(base, CHUNK)])
            return 0

        lax.fori_loop(0, N_CHUNKS, step, 0)

    return emb


_emb = _make_kernel()


@jax.jit
def kernel(x, lut):
    idx = x.reshape(B_TOTAL).astype(jnp.int32)
    out = _emb(idx, lut)
    return out.reshape(BATCH, SEQ, D_MODEL)


# speedup vs baseline: 1.1879x; 1.1879x over previous
import functools

import jax
import jax.numpy as jnp
from jax import lax
from jax.experimental import pallas as pl
from jax.experimental.pallas import tpu as pltpu
from jax.experimental.pallas import tpu_sc as plsc

D_MODEL = 64
MAX_LEN = 16
BATCH = 16384
SEQ = 200

B_TOTAL = BATCH * SEQ
NC, NS = 2, 16
NW = NC * NS
B_PER_W = B_TOTAL // NW
CHUNK = 512
N_CHUNKS = B_PER_W // CHUNK


def _make_kernel():
    mesh = plsc.VectorSubcoreMesh(core_axis_name="c", subcore_axis_name="s")

    @functools.partial(
        pl.kernel,
        mesh=mesh,
        out_type=jax.ShapeDtypeStruct((B_TOTAL, D_MODEL), jnp.float32),
        scratch_types=[
            pltpu.VMEM((CHUNK,), jnp.int32),
            pltpu.VMEM((CHUNK, D_MODEL), jnp.float32),
            pltpu.SemaphoreType.DMA,
        ],
        compiler_params=pltpu.CompilerParams(use_tc_tiling_on_sc=False),
    )
    def emb(x_hbm, lut_hbm, out_hbm, idx_v, rows_v, sem):
        wid = lax.axis_index("s") * NC + lax.axis_index("c")
        base0 = wid * B_PER_W

        def step(i, _):
            base = base0 + i * CHUNK
            pltpu.sync_copy(x_hbm.at[pl.ds(base, CHUNK)], idx_v)
            pltpu.async_copy(lut_hbm.at[idx_v], rows_v, sem).wait()
            pltpu.sync_copy(rows_v, out_hbm.at[pl.ds(base, CHUNK)])
            return 0

        lax.fori_loop(0, N_CHUNKS, step, 0)

    return emb


_emb = _make_kernel()


@jax.jit
def kernel(x, lut):
    idx = x.reshape(B_TOTAL).astype(jnp.int32)
    out = _emb(idx, lut)
    return out.reshape(BATCH, SEQ, D_MODEL)


# speedup vs baseline: 1.1916x; 1.0031x over previous
import functools

import jax
import jax.numpy as jnp
from jax import lax
from jax.experimental import pallas as pl
from jax.experimental.pallas import tpu as pltpu
from jax.experimental.pallas import tpu_sc as plsc

D_MODEL = 64
MAX_LEN = 16
BATCH = 16384
SEQ = 200

B_TOTAL = BATCH * SEQ
NC, NS = 2, 16
NW = NC * NS
B_PER_W = B_TOTAL // NW
CHUNK = 512
N_CHUNKS = B_PER_W // CHUNK


def _make_kernel():
    mesh = plsc.VectorSubcoreMesh(core_axis_name="c", subcore_axis_name="s")

    @functools.partial(
        pl.kernel,
        mesh=mesh,
        out_type=jax.ShapeDtypeStruct((B_TOTAL, D_MODEL), jnp.float32),
        scratch_types=[
            pltpu.VMEM((CHUNK,), jnp.int32),
            pltpu.VMEM((CHUNK,), jnp.int32),
            pltpu.VMEM((CHUNK, D_MODEL), jnp.float32),
            pltpu.VMEM((CHUNK, D_MODEL), jnp.float32),
            pltpu.SemaphoreType.DMA,
            pltpu.SemaphoreType.DMA,
            pltpu.SemaphoreType.DMA,
            pltpu.SemaphoreType.DMA,
            pltpu.SemaphoreType.DMA,
            pltpu.SemaphoreType.DMA,
        ],
        compiler_params=pltpu.CompilerParams(use_tc_tiling_on_sc=False),
    )
    def emb(x_hbm, lut_hbm, out_hbm, idx_v0, idx_v1, rows_v0, rows_v1,
            sin0, sin1, sg0, sg1, so0, so1):
        idx_b = (idx_v0, idx_v1)
        rows_b = (rows_v0, rows_v1)
        sin = (sin0, sin1)
        sg = (sg0, sg1)
        so = (so0, so1)
        wid = lax.axis_index("s") * NC + lax.axis_index("c")
        base0 = wid * B_PER_W

        def load_in(chunk, b):
            pltpu.async_copy(
                x_hbm.at[pl.ds(base0 + chunk * CHUNK, CHUNK)],
                idx_b[b], sin[b])

        def gather(b):
            pltpu.async_copy(lut_hbm.at[idx_b[b]], rows_b[b], sg[b])

        def store_out(chunk, b):
            pltpu.async_copy(
                rows_b[b],
                out_hbm.at[pl.ds(base0 + chunk * CHUNK, CHUNK)], so[b])

        cp_in0 = pltpu.async_copy(
            x_hbm.at[pl.ds(base0, CHUNK)], idx_b[0], sin[0])
        cp_in0.wait()
        pltpu.async_copy(lut_hbm.at[idx_b[0]], rows_b[0], sg[0]).wait()
        load_in(1, 1)

        def group(g, _):
            for b in (0, 1):
                i = g * 2 + b
                q = 1 - b
                store_out(i, b)

                @pl.when(i + 1 < N_CHUNKS)
                def _():
                    pltpu.make_async_copy(
                        x_hbm.at[pl.ds(base0, CHUNK)], idx_b[q], sin[q]
                    ).wait()
                    gather(q)

                pltpu.make_async_copy(
                    rows_b[b],
                    out_hbm.at[pl.ds(base0, CHUNK)], so[b]).wait()

                @pl.when(i + 2 < N_CHUNKS)
                def _():
                    load_in(i + 2, b)

                @pl.when(i + 1 < N_CHUNKS)
                def _():
                    pltpu.make_async_copy(
                        lut_hbm.at[idx_b[q]], rows_b[q], sg[q]).wait()
            return 0

        lax.fori_loop(0, N_CHUNKS // 2, group, 0)

    return emb


_emb = _make_kernel()


@jax.jit
def kernel(x, lut):
    idx = x.reshape(B_TOTAL).astype(jnp.int32)
    out = _emb(idx, lut)
    return out.reshape(BATCH, SEQ, D_MODEL)


# speedup vs baseline: 5.6888x; 4.7740x over previous
import functools

import jax
import jax.numpy as jnp
from jax import lax
from jax.experimental import pallas as pl
from jax.experimental.pallas import tpu as pltpu
from jax.experimental.pallas import tpu_sc as plsc

D_MODEL = 64
MAX_LEN = 16
BATCH = 16384
SEQ = 200

B_TOTAL = BATCH * SEQ
NC, NS = 2, 16
NW = NC * NS
B_PER_W = B_TOTAL // NW
CHUNK = 512
N_CHUNKS = B_PER_W // CHUNK


def _make_kernel():
    mesh = plsc.VectorSubcoreMesh(core_axis_name="c", subcore_axis_name="s")

    @functools.partial(
        pl.kernel,
        mesh=mesh,
        out_type=jax.ShapeDtypeStruct((B_TOTAL, D_MODEL), jnp.float32),
        scratch_types=[
            pltpu.VMEM_SHARED((MAX_LEN, D_MODEL), jnp.float32),
            pltpu.VMEM((CHUNK,), jnp.int32),
            pltpu.VMEM((CHUNK,), jnp.int32),
            pltpu.VMEM((CHUNK, D_MODEL), jnp.float32),
            pltpu.VMEM((CHUNK, D_MODEL), jnp.float32),
            pltpu.SemaphoreType.DMA,
            pltpu.SemaphoreType.DMA,
            pltpu.SemaphoreType.DMA,
            pltpu.SemaphoreType.DMA,
            pltpu.SemaphoreType.DMA,
            pltpu.SemaphoreType.DMA,
        ],
        compiler_params=pltpu.CompilerParams(use_tc_tiling_on_sc=False),
    )
    def emb(x_hbm, lut_hbm, out_hbm, lut_sp, idx_v0, idx_v1, rows_v0, rows_v1,
            sin0, sin1, sg0, sg1, so0, so1):
        idx_b = (idx_v0, idx_v1)
        rows_b = (rows_v0, rows_v1)
        sin = (sin0, sin1)
        sg = (sg0, sg1)
        so = (so0, so1)
        wid = lax.axis_index("s") * NC + lax.axis_index("c")
        base0 = wid * B_PER_W

        @pl.when(lax.axis_index("s") == 0)
        def _():
            pltpu.sync_copy(lut_hbm, lut_sp)

        plsc.subcore_barrier()

        def load_in(chunk, b):
            pltpu.async_copy(
                x_hbm.at[pl.ds(base0 + chunk * CHUNK, CHUNK)],
                idx_b[b], sin[b])

        def gather(b):
            pltpu.async_copy(lut_sp.at[idx_b[b]], rows_b[b], sg[b])

        def store_out(chunk, b):
            pltpu.async_copy(
                rows_b[b],
                out_hbm.at[pl.ds(base0 + chunk * CHUNK, CHUNK)], so[b])

        cp_in0 = pltpu.async_copy(
            x_hbm.at[pl.ds(base0, CHUNK)], idx_b[0], sin[0])
        cp_in0.wait()
        pltpu.async_copy(lut_sp.at[idx_b[0]], rows_b[0], sg[0]).wait()
        load_in(1, 1)

        def group(g, _):
            for b in (0, 1):
                i = g * 2 + b
                q = 1 - b
                store_out(i, b)

                @pl.when(i + 1 < N_CHUNKS)
                def _():
                    pltpu.make_async_copy(
                        x_hbm.at[pl.ds(base0, CHUNK)], idx_b[q], sin[q]
                    ).wait()
                    gather(q)

                pltpu.make_async_copy(
                    rows_b[b],
                    out_hbm.at[pl.ds(base0, CHUNK)], so[b]).wait()

                @pl.when(i + 2 < N_CHUNKS)
                def _():
                    load_in(i + 2, b)

                @pl.when(i + 1 < N_CHUNKS)
                def _():
                    pltpu.make_async_copy(
                        lut_sp.at[idx_b[q]], rows_b[q], sg[q]).wait()
            return 0

        lax.fori_loop(0, N_CHUNKS // 2, group, 0)

    return emb


_emb = _make_kernel()


@jax.jit
def kernel(x, lut):
    idx = x.reshape(B_TOTAL).astype(jnp.int32)
    out = _emb(idx, lut)
    return out.reshape(BATCH, SEQ, D_MODEL)


# speedup vs baseline: 9.4503x; 1.6612x over previous
import functools

import jax
import jax.numpy as jnp
from jax import lax
from jax.experimental import pallas as pl
from jax.experimental.pallas import tpu as pltpu
from jax.experimental.pallas import tpu_sc as plsc

D_MODEL = 64
D_PAD = 128
MAX_LEN = 16
BATCH = 16384
SEQ = 200

B_TOTAL = BATCH * SEQ
NC, NS = 2, 16
NW = NC * NS
B_PER_W = B_TOTAL // NW
CHUNK = 400
N_CHUNKS = B_PER_W // CHUNK


def _make_kernel():
    mesh = plsc.VectorSubcoreMesh(core_axis_name="c", subcore_axis_name="s")

    @functools.partial(
        pl.kernel,
        mesh=mesh,
        out_type=jax.ShapeDtypeStruct((B_TOTAL, D_PAD), jnp.float32),
        scratch_types=[
            pltpu.VMEM_SHARED((MAX_LEN, D_PAD), jnp.float32),
            pltpu.VMEM((CHUNK,), jnp.int32),
            pltpu.VMEM((CHUNK,), jnp.int32),
            pltpu.VMEM((CHUNK, D_PAD), jnp.float32),
            pltpu.VMEM((CHUNK, D_PAD), jnp.float32),
            pltpu.SemaphoreType.DMA,
            pltpu.SemaphoreType.DMA,
            pltpu.SemaphoreType.DMA,
            pltpu.SemaphoreType.DMA,
            pltpu.SemaphoreType.DMA,
            pltpu.SemaphoreType.DMA,
        ],
    )
    def emb(x_hbm, lut_hbm, out_hbm, lut_sp, idx_v0, idx_v1, rows_v0, rows_v1,
            sin0, sin1, sg0, sg1, so0, so1):
        idx_b = (idx_v0, idx_v1)
        rows_b = (rows_v0, rows_v1)
        sin = (sin0, sin1)
        sg = (sg0, sg1)
        so = (so0, so1)
        wid = lax.axis_index("s") * NC + lax.axis_index("c")
        base0 = wid * B_PER_W

        @pl.when(lax.axis_index("s") == 0)
        def _():
            pltpu.sync_copy(lut_hbm, lut_sp)

        plsc.subcore_barrier()

        def load_in(chunk, b):
            pltpu.async_copy(
                x_hbm.at[pl.ds(base0 + chunk * CHUNK, CHUNK)],
                idx_b[b], sin[b])

        def gather(b):
            pltpu.async_copy(lut_sp.at[idx_b[b]], rows_b[b], sg[b])

        def store_out(chunk, b):
            pltpu.async_copy(
                rows_b[b],
                out_hbm.at[pl.ds(base0 + chunk * CHUNK, CHUNK)], so[b])

        pltpu.async_copy(
            x_hbm.at[pl.ds(base0, CHUNK)], idx_b[0], sin[0]).wait()
        pltpu.async_copy(lut_sp.at[idx_b[0]], rows_b[0], sg[0]).wait()
        load_in(1, 1)

        def group(g, _):
            for b in (0, 1):
                i = g * 2 + b
                q = 1 - b
                store_out(i, b)

                @pl.when(i + 1 < N_CHUNKS)
                def _():
                    pltpu.make_async_copy(
                        x_hbm.at[pl.ds(base0, CHUNK)], idx_b[q], sin[q]
                    ).wait()
                    gather(q)

                pltpu.make_async_copy(
                    rows_b[b],
                    out_hbm.at[pl.ds(base0, CHUNK)], so[b]).wait()

                @pl.when(i + 2 < N_CHUNKS)
                def _():
                    load_in(i + 2, b)

                @pl.when(i + 1 < N_CHUNKS)
                def _():
                    pltpu.make_async_copy(
                        lut_sp.at[idx_b[q]], rows_b[q], sg[q]).wait()
            return 0

        lax.fori_loop(0, N_CHUNKS // 2, group, 0)

    return emb


_emb = _make_kernel()


@jax.jit
def kernel(x, lut):
    idx = jnp.minimum(x.reshape(B_TOTAL), MAX_LEN - 1).astype(jnp.int32)
    lut_pad = jnp.pad(lut, ((0, 0), (0, D_PAD - D_MODEL)))
    out = _emb(idx, lut_pad)
    return out[:, :D_MODEL].reshape(BATCH, SEQ, D_MODEL)
